# baseline (speedup 1.0000x reference)
import functools
import math

import jax
import jax.numpy as jnp
from jax import lax
import jax.experimental.pallas as pl
from jax.experimental.pallas import tpu as pltpu
from jax.experimental.pallas import tpu_sc as plsc

B = 32
KV = 8192
HIDDEN = 2048
Q_LORA = 1536
N_HEAD = 32
HEAD_DIM = 128
ROPE_DIM = 64
TOPK = 2048
QDIM = N_HEAD * HEAD_DIM
CK = 8192
INV_SQRT_D = 1.0 / math.sqrt(float(HEAD_DIM))
NEG = -1e30

_P = jax.lax.Precision.DEFAULT
_ISOLATE = 0


def _dot(a, b):
    return jax.lax.dot_general(a, b, (((1,), (0,)), ((), ())),
                               preferred_element_type=jnp.float32, precision=_P)


def _dot_t(a, b):
    return jax.lax.dot_general(a, b, (((1,), (1,)), ((), ())),
                               preferred_element_type=jnp.float32, precision=_P)


def _prologue_body(qr_ref, wqb_ref, hid_ref, wk_ref, g_ref, be_ref, ww_ref,
                   bw_ref, cq_ref, sq_ref, ck_ref, sk_ref, e_ref, s_ref,
                   q_out, k_out, w_out, fix_out):
    q2 = _dot(qr_ref[...], wqb_ref[...])
    lane_q = jax.lax.broadcasted_iota(jnp.int32, (B, QDIM), 1) % HEAD_DIM
    rot_q = jnp.where(lane_q < 32, -pltpu.roll(q2, QDIM - 32, 1),
                      pltpu.roll(q2, 32, 1))
    q2 = q2 * cq_ref[...] + rot_q * sq_ref[...]
    q_out[...] = q2

    kp = _dot(hid_ref[...], wk_ref[...])
    mu = jnp.mean(kp, axis=1, keepdims=True)
    var = jnp.mean((kp - mu) ** 2, axis=1, keepdims=True)
    kn = (kp - mu) / jnp.sqrt(var + 1e-6) * g_ref[...] + be_ref[...]
    lane_k = jax.lax.broadcasted_iota(jnp.int32, (B, HEAD_DIM), 1)
    rot_k = jnp.where(lane_k < 32, -pltpu.roll(kn, HEAD_DIM - 32, 1),
                      pltpu.roll(kn, 32, 1))
    kn = kn * ck_ref[...] + rot_k * sk_ref[...]
    k_out[...] = kn

    w = _dot(hid_ref[...], ww_ref[...]) + bw_ref[...]
    w_out[...] = w

    qsel = _dot(e_ref[...], q2)
    knb = kn.astype(jnp.bfloat16).astype(jnp.float32)
    ktile = jnp.tile(knb, (1, N_HEAD))
    l32 = jax.lax.dot_general(qsel * ktile, s_ref[...],
                              (((1,), (0,)), ((), ())),
                              preferred_element_type=jnp.float32,
                              precision=jax.lax.Precision.HIGHEST) * INV_SQRT_D
    wsel = _dot(e_ref[...], w)
    r32 = jnp.maximum(l32, 0.0).astype(jnp.bfloat16).astype(jnp.float32)
    fix_out[...] = jnp.sum(wsel * r32, axis=1, keepdims=True)


def _run_prologue(qr, Wq_b, hidden, Wk, gamma, beta, W_weights, b_weights,
                  cos, sin, slot_b):
    cq = jnp.tile(jnp.concatenate([cos, jnp.ones((B, ROPE_DIM), jnp.float32)],
                                  axis=1), (1, N_HEAD))
    sq = jnp.tile(jnp.concatenate([sin, jnp.zeros((B, ROPE_DIM), jnp.float32)],
                                  axis=1), (1, N_HEAD))
    ck = jnp.concatenate([cos, jnp.ones((B, ROPE_DIM), jnp.float32)], axis=1)
    sk = jnp.concatenate([sin, jnp.zeros((B, ROPE_DIM), jnp.float32)], axis=1)
    e = (slot_b[:, None] == jnp.arange(B, dtype=jnp.int32)[None, :])
    e = e.astype(jnp.float32)
    s = (jnp.arange(QDIM, dtype=jnp.int32)[:, None] // HEAD_DIM
         == jnp.arange(N_HEAD, dtype=jnp.int32)[None, :]).astype(jnp.float32)
    return pl.pallas_call(
        _prologue_body,
        out_shape=(
            jax.ShapeDtypeStruct((B, QDIM), jnp.float32),
            jax.ShapeDtypeStruct((B, HEAD_DIM), jnp.float32),
            jax.ShapeDtypeStruct((B, N_HEAD), jnp.float32),
            jax.ShapeDtypeStruct((B, 1), jnp.float32),
        ),
    )(qr, Wq_b, hidden, Wk, gamma.reshape(1, HEAD_DIM),
      beta.reshape(1, HEAD_DIM), W_weights, b_weights.reshape(1, N_HEAD),
      cq, sq, ck, sk, e, s)


def _score_body_off(b0, q_ref, cache_ref, w_ref, seq_ref, out_ref):
    b = pl.program_id(0)
    c = pl.program_id(1)
    l = _dot_t(q_ref[...], cache_ref[...]) * INV_SQRT_D
    r = jnp.maximum(l, 0.0)
    s = _dot(w_ref[0], r)
    pos = c * CK + jax.lax.broadcasted_iota(jnp.int32, (1, CK), 1)
    out_ref[0] = jnp.where(pos < seq_ref[b0 + b], s, NEG)


def _run_score(q_flat, kv_cache, w, seq_lens, b0, nb):
    nc = KV // CK
    out = pl.pallas_call(
        functools.partial(_score_body_off, b0),
        grid=(nb, nc),
        in_specs=[
            pl.BlockSpec((N_HEAD, HEAD_DIM), lambda b, c: (b0 + b, 0)),
            pl.BlockSpec((CK, HEAD_DIM),
                         lambda b, c: ((b0 + b) * (KV // CK) + c, 0)),
            pl.BlockSpec((1, 1, N_HEAD), lambda b, c: (b0 + b, 0, 0)),
            pl.BlockSpec((B,), lambda b, c: (0,),
                         memory_space=pltpu.SMEM),
        ],
        out_specs=pl.BlockSpec((1, 1, CK), lambda b, c: (b * (KV // CK) + c, 0, 0)),
        out_shape=jax.ShapeDtypeStruct((nb * nc, 1, CK), jnp.float32),
    )(q_flat, kv_cache, w.reshape(B, 1, N_HEAD), seq_lens)
    return out.reshape(nb, KV)


_NCHUNK = KV // 16
_OCHUNK = TOPK // 16
_NBIN = 256
_SELCAP = KV + 16

@functools.lru_cache(maxsize=4)
def _get_sc_topk(r0, nrows):
    mesh = plsc.VectorSubcoreMesh(core_axis_name="c", subcore_axis_name="s",
                                  num_cores=2, num_subcores=16)
    return functools.partial(
        pl.kernel,
        out_type=(jax.ShapeDtypeStruct((nrows, TOPK), jnp.float32),
                  jax.ShapeDtypeStruct((nrows, TOPK), jnp.int32)),
        mesh=mesh,
        scratch_types=[
        pltpu.VMEM((KV,), jnp.float32),
        pltpu.VMEM((KV,), jnp.int32),
        pltpu.VMEM((_SELCAP,), jnp.int32),
        pltpu.VMEM((_SELCAP,), jnp.int32),
        pltpu.VMEM((_SELCAP,), jnp.int32),
        pltpu.VMEM((_SELCAP,), jnp.int32),
        pltpu.VMEM((TOPK,), jnp.float32),
        pltpu.VMEM((_NBIN,), jnp.int32),
        pltpu.VMEM((_NBIN,), jnp.int32),
        pltpu.VMEM((16,), jnp.int32),
        pltpu.VMEM((B,), jnp.int32),
        pltpu.VMEM((B,), jnp.int32),
        pltpu.VMEM((B,), jnp.float32),
            pltpu.VMEM((B,), jnp.int32),
        ],
        compiler_params=pltpu.CompilerParams(needs_layout_passes=False),
    )(functools.partial(_sc_topk_body, r0, nrows))


def _sc_topk_body(r0, nrows, score_hbm, sb_hbm, sp_hbm, fix_hbm, seq_hbm,
             tv_hbm, ti_hbm,
             row, keys, sel_ka, sel_kb, sel_ia, sel_ib, vals, hist, pref,
             buf16, sbv, spv, fxv, sqv):
    wid = lax.axis_index("s") * 2 + lax.axis_index("c")

    @pl.when(wid < nrows)
    def _row_work():
        _sc_topk_row(r0, wid, score_hbm, sb_hbm, sp_hbm, fix_hbm, seq_hbm,
                     tv_hbm, ti_hbm, row, keys, sel_ka, sel_kb, sel_ia,
                     sel_ib, vals, hist, pref, buf16, sbv, spv, fxv, sqv)


def _sc_topk_row(r0, wid, score_hbm, sb_hbm, sp_hbm, fix_hbm, seq_hbm,
             tv_hbm, ti_hbm,
             row, keys, sel_ka, sel_kb, sel_ia, sel_ib, vals, hist, pref,
             buf16, sbv, spv, fxv, sqv):
    pltpu.sync_copy(score_hbm.at[wid], row)
    pltpu.sync_copy(sb_hbm, sbv)
    pltpu.sync_copy(sp_hbm, spv)
    pltpu.sync_copy(fix_hbm, fxv)
    pltpu.sync_copy(seq_hbm, sqv)

    iota16 = lax.iota(jnp.int32, 16)
    zeros16 = jnp.zeros((16,), jnp.int32)
    last16 = jnp.full((16,), 15, jnp.int32)
    grow_vec = jnp.full((16,), r0 + wid, jnp.int32)

    seq_r = plsc.load_gather(sqv, [grow_vec])
    for c2 in range(B // 16):
        sl = pl.ds(c2 * 16, 16)
        sp_c = spv[sl]
        m = (sbv[sl] == grow_vec) & (sp_c < seq_r)
        plsc.store_scatter(row, [sp_c], fxv[sl], mask=m)

    def _zero_hist():
        for t in range(_NBIN // 16):
            hist[pl.ds(t * 16, 16)] = zeros16

    def _prefix_hist():
        carry = zeros16
        for t in range(_NBIN // 16):
            sl = pl.ds(t * 16, 16)
            cs = plsc.cumsum(hist[sl]) + carry
            pref[sl] = cs
            buf16[...] = cs
            carry = plsc.load_gather(buf16, [last16])

    def _find_cut(target_vec):
        cut = jnp.int32(_NBIN - 1)
        for t in range(_NBIN // 16):
            sl = pl.ds(t * 16, 16)
            found = pref[sl] >= target_vec
            cand = jnp.where(found, t * 16 + iota16, _NBIN)
            cut = jnp.minimum(cut, jnp.min(cand))
        cut_vec = jnp.full((16,), cut, jnp.int32)
        below = plsc.load_gather(pref, [jnp.maximum(cut_vec - 1, 0)])
        below = jnp.where(cut_vec == 0, 0, below)
        return cut_vec, below

    _zero_hist()

    @plsc.parallel_loop(0, _NCHUNK, unroll=4)
    def _build(j):
        o = pl.multiple_of(j * 16, 16)
        sl = pl.ds(o, 16)
        u = plsc.bitcast(row[sl], jnp.uint32)
        neg = (u >> jnp.uint32(31)) == jnp.uint32(1)
        dk = jnp.where(neg, u, (~u) & jnp.uint32(0x7FFFFFFF))
        k = plsc.bitcast(dk, jnp.int32)
        keys[sl] = k
        d = lax.shift_right_logical(k, 24)
        cnt, lastm = plsc.scan_count(d)
        plsc.addupdate_scatter(hist, [d], cnt, mask=lastm)
    _prefix_hist()
    target = jnp.full((16,), TOPK, jnp.int32)
    cut1, below1 = _find_cut(target)

    _zero_hist()

    @plsc.parallel_loop(0, _NCHUNK, unroll=4)
    def _hist2(j):
        o = pl.multiple_of(j * 16, 16)
        k = keys[pl.ds(o, 16)]
        m = lax.shift_right_logical(k, 24) == cut1
        d = lax.shift_right_logical(k, 16) & 255
        cnt, lastm = plsc.scan_count(d, mask=m)
        plsc.addupdate_scatter(hist, [d], cnt, mask=lastm)
    _prefix_hist()
    cut2, _ = _find_cut(target - below1)
    sel16 = cut1 * 256 + cut2

    def _compact(j, ctr):
        o = pl.multiple_of(j * 16, 16)
        k = keys[pl.ds(o, 16)]
        m = lax.shift_right_logical(k, 16) <= sel16
        cs = plsc.cumsum(m.astype(jnp.int32))
        pos = ctr + cs - 1
        plsc.store_scatter(sel_kb, [pos], k, mask=m)
        plsc.store_scatter(sel_ib, [pos], j * 16 + iota16, mask=m)
        return ctr + plsc.all_reduce_population_count(m)

    s_vec = lax.fori_loop(0, _NCHUNK, _compact, zeros16)
    pad_pos = s_vec + iota16
    plsc.store_scatter(sel_kb, [pad_pos], jnp.full((16,), -1, jnp.int32),
                       mask=pad_pos < _SELCAP)
    s = jnp.max(s_vec)
    nch = lax.shift_right_logical(s + 15, 4)

    for p in range(4):
        src_k, dst_k = (sel_kb, sel_ka) if p % 2 == 0 else (sel_ka, sel_kb)
        src_i, dst_i = (sel_ib, sel_ia) if p % 2 == 0 else (sel_ia, sel_ib)
        shift = jnp.int32(8 * p)

        _zero_hist()

        @plsc.parallel_loop(0, nch, unroll=4)
        def _hist(j):
            o = pl.multiple_of(j * 16, 16)
            d = lax.shift_right_logical(src_k[pl.ds(o, 16)], shift) & 255
            cnt, lastm = plsc.scan_count(d)
            plsc.addupdate_scatter(hist, [d], cnt, mask=lastm)

        carry = zeros16
        for t in range(_NBIN // 16):
            sl = pl.ds(t * 16, 16)
            h = hist[sl]
            cs = plsc.cumsum(h)
            hist[sl] = cs - h + carry
            buf16[...] = cs
            carry = carry + plsc.load_gather(buf16, [last16])

        def _perm(j, _):
            o = pl.multiple_of(j * 16, 16)
            sl = pl.ds(o, 16)
            k = src_k[sl]
            v = src_i[sl]
            d = lax.shift_right_logical(k, shift) & 255
            cnt, lastm = plsc.scan_count(d)
            base = plsc.load_gather(hist, [d])
            pos = base + cnt - 1
            plsc.store_scatter(dst_k, [pos], k)
            plsc.store_scatter(dst_i, [pos], v)
            plsc.addupdate_scatter(hist, [d], cnt, mask=lastm)
            return 0

        lax.fori_loop(0, nch, _perm, 0)

    @plsc.parallel_loop(0, _OCHUNK, unroll=4)
    def _out(j):
        o = pl.multiple_of(j * 16, 16)
        sl = pl.ds(o, 16)
        vals[sl] = plsc.load_gather(row, [sel_ib[sl]])
    pltpu.sync_copy(vals, tv_hbm.at[wid])
    pltpu.sync_copy(sel_ib.at[pl.ds(0, TOPK)], ti_hbm.at[wid])


def kernel(hidden_states, qr, cos, sin, kv_cache, slot_mapping, seq_lens,
           Wq_b, Wk, k_norm_gamma, k_norm_beta, W_weights, b_weights):
    slot_b = (slot_mapping // KV).astype(jnp.int32)
    slot_p = (slot_mapping % KV).astype(jnp.int32)

    q2, k_new, w, fix = _run_prologue(qr, Wq_b, hidden_states, Wk,
                                      k_norm_gamma, k_norm_beta, W_weights,
                                      b_weights, cos, sin, slot_b)
    q_flat = q2.reshape(B, N_HEAD, HEAD_DIM).reshape(B * N_HEAD, HEAD_DIM)
    fixv = fix.reshape(B)
    half = B // 2
    score0 = _run_score(q_flat, kv_cache, w, seq_lens, 0, half)
    tv0, ti0 = _get_sc_topk(0, half)(score0, slot_b, slot_p, fixv, seq_lens)
    score1 = _run_score(q_flat, kv_cache, w, seq_lens, half, half)
    tv1, ti1 = _get_sc_topk(half, half)(score1, slot_b, slot_p, fixv, seq_lens)
    top_vals = jnp.concatenate([tv0, tv1], axis=0)
    top_idx = jnp.concatenate([ti0, ti1], axis=0)
    return top_vals, top_idx, k_new


# speedup vs baseline: 1.0941x; 1.0941x over previous
import functools
import math

import jax
import jax.numpy as jnp
from jax import lax
import jax.experimental.pallas as pl
from jax.experimental.pallas import tpu as pltpu
from jax.experimental.pallas import tpu_sc as plsc

B = 32
KV = 8192
HIDDEN = 2048
Q_LORA = 1536
N_HEAD = 32
HEAD_DIM = 128
ROPE_DIM = 64
TOPK = 2048
QDIM = N_HEAD * HEAD_DIM
CK = 8192
INV_SQRT_D = 1.0 / math.sqrt(float(HEAD_DIM))
NEG = -1e30

_P = jax.lax.Precision.DEFAULT


def _dot(a, b):
    return jax.lax.dot_general(a, b, (((1,), (0,)), ((), ())),
                               preferred_element_type=jnp.float32, precision=_P)


def _dot_t(a, b):
    return jax.lax.dot_general(a, b, (((1,), (1,)), ((), ())),
                               preferred_element_type=jnp.float32, precision=_P)


def _prologue_body(qr_ref, wqb_ref, hid_ref, wk_ref, g_ref, be_ref, ww_ref,
                   bw_ref, cq_ref, sq_ref, ck_ref, sk_ref, e_ref, s_ref,
                   q_out, k_out, w_out, fix_out):
    q2 = _dot(qr_ref[...], wqb_ref[...])
    lane_q = jax.lax.broadcasted_iota(jnp.int32, (B, QDIM), 1) % HEAD_DIM
    rot_q = jnp.where(lane_q < 32, -pltpu.roll(q2, QDIM - 32, 1),
                      pltpu.roll(q2, 32, 1))
    q2 = q2 * cq_ref[...] + rot_q * sq_ref[...]
    q_out[...] = q2

    kp = _dot(hid_ref[...], wk_ref[...])
    mu = jnp.mean(kp, axis=1, keepdims=True)
    var = jnp.mean((kp - mu) ** 2, axis=1, keepdims=True)
    kn = (kp - mu) / jnp.sqrt(var + 1e-6) * g_ref[...] + be_ref[...]
    lane_k = jax.lax.broadcasted_iota(jnp.int32, (B, HEAD_DIM), 1)
    rot_k = jnp.where(lane_k < 32, -pltpu.roll(kn, HEAD_DIM - 32, 1),
                      pltpu.roll(kn, 32, 1))
    kn = kn * ck_ref[...] + rot_k * sk_ref[...]
    k_out[...] = kn

    w = _dot(hid_ref[...], ww_ref[...]) + bw_ref[...]
    w_out[...] = w

    qsel = _dot(e_ref[...], q2)
    knb = kn.astype(jnp.bfloat16).astype(jnp.float32)
    ktile = jnp.tile(knb, (1, N_HEAD))
    l32 = jax.lax.dot_general(qsel * ktile, s_ref[...],
                              (((1,), (0,)), ((), ())),
                              preferred_element_type=jnp.float32,
                              precision=jax.lax.Precision.HIGHEST) * INV_SQRT_D
    wsel = _dot(e_ref[...], w)
    r32 = jnp.maximum(l32, 0.0).astype(jnp.bfloat16).astype(jnp.float32)
    fix_out[...] = jnp.sum(wsel * r32, axis=1, keepdims=True)


def _run_prologue(qr, Wq_b, hidden, Wk, gamma, beta, W_weights, b_weights,
                  cos, sin, slot_b):
    cq = jnp.tile(jnp.concatenate([cos, jnp.ones((B, ROPE_DIM), jnp.float32)],
                                  axis=1), (1, N_HEAD))
    sq = jnp.tile(jnp.concatenate([sin, jnp.zeros((B, ROPE_DIM), jnp.float32)],
                                  axis=1), (1, N_HEAD))
    ck = jnp.concatenate([cos, jnp.ones((B, ROPE_DIM), jnp.float32)], axis=1)
    sk = jnp.concatenate([sin, jnp.zeros((B, ROPE_DIM), jnp.float32)], axis=1)
    e = (slot_b[:, None] == jnp.arange(B, dtype=jnp.int32)[None, :])
    e = e.astype(jnp.float32)
    s = (jnp.arange(QDIM, dtype=jnp.int32)[:, None] // HEAD_DIM
         == jnp.arange(N_HEAD, dtype=jnp.int32)[None, :]).astype(jnp.float32)
    return pl.pallas_call(
        _prologue_body,
        out_shape=(
            jax.ShapeDtypeStruct((B, QDIM), jnp.float32),
            jax.ShapeDtypeStruct((B, HEAD_DIM), jnp.float32),
            jax.ShapeDtypeStruct((B, N_HEAD), jnp.float32),
            jax.ShapeDtypeStruct((B, 1), jnp.float32),
        ),
    )(qr, Wq_b, hidden, Wk, gamma.reshape(1, HEAD_DIM),
      beta.reshape(1, HEAD_DIM), W_weights, b_weights.reshape(1, N_HEAD),
      cq, sq, ck, sk, e, s)


def _score_body_off(b0, q_ref, cache_ref, w_ref, seq_ref, out_ref):
    b = pl.program_id(0)
    c = pl.program_id(1)
    l = _dot_t(q_ref[...], cache_ref[...]) * INV_SQRT_D
    r = jnp.maximum(l, 0.0)
    s = _dot(w_ref[0], r)
    pos = c * CK + jax.lax.broadcasted_iota(jnp.int32, (1, CK), 1)
    out_ref[0] = jnp.where(pos < seq_ref[b0 + b], s, NEG)


def _run_score(q_flat, kv_cache, w, seq_lens, b0, nb):
    nc = KV // CK
    out = pl.pallas_call(
        functools.partial(_score_body_off, b0),
        grid=(nb, nc),
        in_specs=[
            pl.BlockSpec((N_HEAD, HEAD_DIM), lambda b, c: (b0 + b, 0)),
            pl.BlockSpec((CK, HEAD_DIM),
                         lambda b, c: ((b0 + b) * (KV // CK) + c, 0)),
            pl.BlockSpec((1, 1, N_HEAD), lambda b, c: (b0 + b, 0, 0)),
            pl.BlockSpec((B,), lambda b, c: (0,),
                         memory_space=pltpu.SMEM),
        ],
        out_specs=pl.BlockSpec((1, 1, CK), lambda b, c: (b * (KV // CK) + c, 0, 0)),
        out_shape=jax.ShapeDtypeStruct((nb * nc, 1, CK), jnp.float32),
    )(q_flat, kv_cache, w.reshape(B, 1, N_HEAD), seq_lens)
    return out.reshape(nb, KV)


_NCHUNK = KV // 16
_OCHUNK = TOPK // 16
_NBIN = 256
_SELCAP = KV + 16

@functools.lru_cache(maxsize=4)
def _get_sc_topk(r0, nrows):
    mesh = plsc.VectorSubcoreMesh(core_axis_name="c", subcore_axis_name="s",
                                  num_cores=2, num_subcores=16)
    return functools.partial(
        pl.kernel,
        out_type=(jax.ShapeDtypeStruct((nrows, TOPK), jnp.float32),
                  jax.ShapeDtypeStruct((nrows, TOPK), jnp.int32)),
        mesh=mesh,
        scratch_types=[
        pltpu.VMEM((KV,), jnp.float32),
        pltpu.VMEM((KV,), jnp.int32),
        pltpu.VMEM((_SELCAP,), jnp.int32),
        pltpu.VMEM((_SELCAP,), jnp.int32),
        pltpu.VMEM((_SELCAP,), jnp.int32),
        pltpu.VMEM((_SELCAP,), jnp.int32),
        pltpu.VMEM((TOPK,), jnp.float32),
        pltpu.VMEM((_NBIN,), jnp.int32),
        pltpu.VMEM((_NBIN,), jnp.int32),
        pltpu.VMEM((16,), jnp.int32),
        pltpu.VMEM((B,), jnp.int32),
        pltpu.VMEM((B,), jnp.int32),
        pltpu.VMEM((B,), jnp.float32),
            pltpu.VMEM((B,), jnp.int32),
        ],
        compiler_params=pltpu.CompilerParams(needs_layout_passes=False),
    )(functools.partial(_sc_topk_body, r0, nrows))


def _sc_topk_body(r0, nrows, score_hbm, sb_hbm, sp_hbm, fix_hbm, seq_hbm,
             tv_hbm, ti_hbm,
             row, keys, sel_ka, sel_kb, sel_ia, sel_ib, vals, hist, pref,
             buf16, sbv, spv, fxv, sqv):
    wid = lax.axis_index("s") * 2 + lax.axis_index("c")

    @pl.when(wid < nrows)
    def _row_work():
        _sc_topk_row(r0, wid, score_hbm, sb_hbm, sp_hbm, fix_hbm, seq_hbm,
                     tv_hbm, ti_hbm, row, keys, sel_ka, sel_kb, sel_ia,
                     sel_ib, vals, hist, pref, buf16, sbv, spv, fxv, sqv)


def _sc_topk_row(r0, wid, score_hbm, sb_hbm, sp_hbm, fix_hbm, seq_hbm,
             tv_hbm, ti_hbm,
             row, keys, sel_ka, sel_kb, sel_ia, sel_ib, vals, hist, pref,
             buf16, sbv, spv, fxv, sqv):
    pltpu.sync_copy(score_hbm.at[wid], row)
    pltpu.sync_copy(sb_hbm, sbv)
    pltpu.sync_copy(sp_hbm, spv)
    pltpu.sync_copy(fix_hbm, fxv)
    pltpu.sync_copy(seq_hbm, sqv)

    iota16 = lax.iota(jnp.int32, 16)
    zeros16 = jnp.zeros((16,), jnp.int32)
    last16 = jnp.full((16,), 15, jnp.int32)
    grow_vec = jnp.full((16,), r0 + wid, jnp.int32)

    seq_r = plsc.load_gather(sqv, [grow_vec])
    for c2 in range(B // 16):
        sl = pl.ds(c2 * 16, 16)
        sp_c = spv[sl]
        m = (sbv[sl] == grow_vec) & (sp_c < seq_r)
        plsc.store_scatter(row, [sp_c], fxv[sl], mask=m)

    def _zero_hist():
        for t in range(_NBIN // 16):
            hist[pl.ds(t * 16, 16)] = zeros16

    def _prefix_hist():
        carry = zeros16
        for t in range(_NBIN // 16):
            sl = pl.ds(t * 16, 16)
            cs = plsc.cumsum(hist[sl]) + carry
            pref[sl] = cs
            buf16[...] = cs
            carry = plsc.load_gather(buf16, [last16])

    def _find_cut(target_vec):
        cut = jnp.int32(_NBIN - 1)
        for t in range(_NBIN // 16):
            sl = pl.ds(t * 16, 16)
            found = pref[sl] >= target_vec
            cand = jnp.where(found, t * 16 + iota16, _NBIN)
            cut = jnp.minimum(cut, jnp.min(cand))
        cut_vec = jnp.full((16,), cut, jnp.int32)
        below = plsc.load_gather(pref, [jnp.maximum(cut_vec - 1, 0)])
        below = jnp.where(cut_vec == 0, 0, below)
        return cut_vec, below

    _zero_hist()

    @plsc.parallel_loop(0, _NCHUNK, unroll=4)
    def _build(j):
        o = pl.multiple_of(j * 16, 16)
        sl = pl.ds(o, 16)
        u = plsc.bitcast(row[sl], jnp.uint32)
        neg = (u >> jnp.uint32(31)) == jnp.uint32(1)
        dk = jnp.where(neg, u, (~u) & jnp.uint32(0x7FFFFFFF))
        k = plsc.bitcast(dk, jnp.int32)
        keys[sl] = k
        d = lax.shift_right_logical(k, 24)
        cnt, lastm = plsc.scan_count(d)
        plsc.addupdate_scatter(hist, [d], cnt, mask=lastm)
    _prefix_hist()
    target = jnp.full((16,), TOPK, jnp.int32)
    cut1, below1 = _find_cut(target)

    _zero_hist()

    @plsc.parallel_loop(0, _NCHUNK, unroll=4)
    def _hist2(j):
        o = pl.multiple_of(j * 16, 16)
        k = keys[pl.ds(o, 16)]
        m = lax.shift_right_logical(k, 24) == cut1
        d = lax.shift_right_logical(k, 16) & 255
        cnt, lastm = plsc.scan_count(d, mask=m)
        plsc.addupdate_scatter(hist, [d], cnt, mask=lastm)
    _prefix_hist()
    cut2, _ = _find_cut(target - below1)
    sel16 = cut1 * 256 + cut2

    def _compact(j, ctr):
        o = pl.multiple_of(j * 16, 16)
        k = keys[pl.ds(o, 16)]
        m = lax.shift_right_logical(k, 16) <= sel16
        cs = plsc.cumsum(m.astype(jnp.int32))
        pos = ctr + cs - 1
        plsc.store_scatter(sel_kb, [pos], k, mask=m)
        plsc.store_scatter(sel_ib, [pos], j * 16 + iota16, mask=m)
        return ctr + plsc.all_reduce_population_count(m)

    s_vec = lax.fori_loop(0, _NCHUNK, _compact, zeros16)
    pad_pos = s_vec + iota16
    plsc.store_scatter(sel_kb, [pad_pos], jnp.full((16,), -1, jnp.int32),
                       mask=pad_pos < _SELCAP)
    s = jnp.max(s_vec)
    nch = lax.shift_right_logical(s + 15, 4)

    for p in range(4):
        src_k, dst_k = (sel_kb, sel_ka) if p % 2 == 0 else (sel_ka, sel_kb)
        src_i, dst_i = (sel_ib, sel_ia) if p % 2 == 0 else (sel_ia, sel_ib)
        shift = jnp.int32(8 * p)

        _zero_hist()

        @plsc.parallel_loop(0, nch, unroll=4)
        def _hist(j):
            o = pl.multiple_of(j * 16, 16)
            d = lax.shift_right_logical(src_k[pl.ds(o, 16)], shift) & 255
            cnt, lastm = plsc.scan_count(d)
            plsc.addupdate_scatter(hist, [d], cnt, mask=lastm)

        carry = zeros16
        for t in range(_NBIN // 16):
            sl = pl.ds(t * 16, 16)
            h = hist[sl]
            cs = plsc.cumsum(h)
            hist[sl] = cs - h + carry
            buf16[...] = cs
            carry = carry + plsc.load_gather(buf16, [last16])

        def _perm(j, _):
            o = pl.multiple_of(j * 16, 16)
            sl = pl.ds(o, 16)
            k = src_k[sl]
            v = src_i[sl]
            d = lax.shift_right_logical(k, shift) & 255
            cnt, lastm = plsc.scan_count(d)
            base = plsc.load_gather(hist, [d])
            pos = base + cnt - 1
            plsc.store_scatter(dst_k, [pos], k)
            plsc.store_scatter(dst_i, [pos], v)
            plsc.addupdate_scatter(hist, [d], cnt, mask=lastm)
            return 0

        lax.fori_loop(0, nch, _perm, 0)

    @plsc.parallel_loop(0, _OCHUNK, unroll=4)
    def _out(j):
        o = pl.multiple_of(j * 16, 16)
        sl = pl.ds(o, 16)
        vals[sl] = plsc.load_gather(row, [sel_ib[sl]])
    pltpu.sync_copy(vals, tv_hbm.at[wid])
    pltpu.sync_copy(sel_ib.at[pl.ds(0, TOPK)], ti_hbm.at[wid])


def kernel(hidden_states, qr, cos, sin, kv_cache, slot_mapping, seq_lens,
           Wq_b, Wk, k_norm_gamma, k_norm_beta, W_weights, b_weights):
    slot_b = (slot_mapping // KV).astype(jnp.int32)
    slot_p = (slot_mapping % KV).astype(jnp.int32)

    q2, k_new, w, fix = _run_prologue(qr, Wq_b, hidden_states, Wk,
                                      k_norm_gamma, k_norm_beta, W_weights,
                                      b_weights, cos, sin, slot_b)
    q_flat = q2.reshape(B, N_HEAD, HEAD_DIM).reshape(B * N_HEAD, HEAD_DIM)
    fixv = fix.reshape(B)
    score = _run_score(q_flat, kv_cache, w, seq_lens, 0, B)
    top_vals, top_idx = _get_sc_topk(0, B)(score, slot_b, slot_p, fixv,
                                           seq_lens)
    return top_vals, top_idx, k_new
